# baseline (speedup 1.0000x reference)
import functools

import jax
import jax.numpy as jnp
from jax import lax
from jax.experimental import pallas as pl
from jax.experimental.pallas import tpu as pltpu
from jax.experimental.pallas import tpu_sc as plsc

_NC = 2
_NS = 16
_NW = _NC * _NS
_L = 16


def _prep_body(x_ref, xh_ref, nrm_ref):
    x = x_ref[...]
    ss = jnp.sum(x * x, axis=1, keepdims=True)
    r = lax.rsqrt(ss)
    xh_ref[...] = x * r
    nrm_ref[...] = (ss * r)[:, 0]


def _final_body(k_inv, d, s_ref, ws_ref, x_ref, w_ref, b_ref, o_ref):
    wn = w_ref[:, :d]
    wc = w_ref[:, d:]
    dn = (((1,), (1,)), ((), ()))
    ctr = lax.dot_general(x_ref[...], wc, dn,
                          preferred_element_type=jnp.float32) + b_ref[...]
    sn = lax.dot_general(s_ref[...], wn, dn,
                         preferred_element_type=jnp.float32)
    o_ref[...] = (sn + ws_ref[...] * ctr) * k_inv


def _make_sc_gather(np_, c, k, d):
    mesh = plsc.VectorSubcoreMesh(core_axis_name="c", subcore_axis_name="s")
    nsub = d // _L
    kw = (k + 1 + 7) // 8 * 8
    ng = (k + 1 + _L - 1) // _L
    G = 2
    span = G * kw
    glen = (G - 1) * kw + k + 1
    NBUF = 3
    nblk = c // G

    def body(xh_hbm, idx_hbm, nrm_hbm, s_hbm, ws_hbm,
             idx_v, nrm_v, b0, b1, b2, s_v, ws_v, sem0, sem1, sem2):
        bufs = (b0, b1, b2)
        sems = (sem0, sem1, sem2)
        wid = lax.axis_index("s") * _NC + lax.axis_index("c")
        base = wid * c
        pltpu.sync_copy(idx_hbm.at[pl.ds(base * kw, c * kw)],
                        idx_v.at[pl.ds(0, c * kw)])
        idx_v[pl.ds(c * kw, _L)] = jnp.zeros((_L,), jnp.int32)
        pltpu.sync_copy(nrm_hbm, nrm_v)
        lane = lax.iota(jnp.int32, _L)

        def start(blk, buf, sem):
            pltpu.async_copy(
                xh_hbm.at[idx_v.at[pl.ds(blk * span, glen)]], buf, sem)

        def wait(buf, sem):
            pltpu.make_async_copy(
                xh_hbm.at[idx_v.at[pl.ds(0, glen)]], buf, sem).wait()

        def compute(blk, buf):
            ib = blk * span
            for g in range(G):
                r0 = g * kw
                xn = [buf[r0 + k, pl.ds(t * _L, _L)] for t in range(nsub)]
                acc = [jnp.zeros((_L,), jnp.float32) for _ in range(nsub)]
                ws = jnp.float32(0.0)
                for grp in range(k // _L):
                    rng = plsc.load_gather(
                        nrm_v, [idx_v[pl.ds(ib + g * kw + grp * _L, _L)]])
                    rb = r0 + grp * _L

                    def edge(e2, carry):
                        *acc2, ws2 = carry
                        row = [buf[rb + e2, pl.ds(t * _L, _L)]
                               for t in range(nsub)]
                        dv = row[0] * xn[0]
                        for t in range(1, nsub):
                            dv = dv + row[t] * xn[t]
                        w = jnp.sum(dv)
                        wn = w * rng.at[jnp.full((_L,), e2, jnp.int32)].get(
                            mode="promise_in_bounds")
                        return tuple(
                            [acc2[t] + wn * row[t] for t in range(nsub)]
                            + [ws2 + w])
                    *acc, ws = lax.fori_loop(0, _L, edge, (*acc, ws),
                                             unroll=2)
                i = blk * G + g
                for t in range(nsub):
                    s_v[i, pl.ds(t * _L, _L)] = acc[t]
                plsc.store_scatter(ws_v, [jnp.full((_L,), i, jnp.int32)],
                                   jnp.full((_L,), ws, jnp.float32),
                                   mask=lane == 0)

        def blk_body(blk, carry):
            pltpu.async_copy(
                xh_hbm.at[idx_v.at[pl.ds(blk * span, glen)]],
                bufs[0], sems[0]).wait()
            compute(blk, bufs[0])
            return carry

        lax.fori_loop(0, nblk, blk_body, 0, unroll=False)
        pltpu.sync_copy(s_v, s_hbm.at[pl.ds(base, c)])
        pltpu.sync_copy(ws_v, ws_hbm.at[pl.ds(base, c)])

    return pl.kernel(
        body,
        out_type=[
            jax.ShapeDtypeStruct((np_, d), jnp.float32),
            jax.ShapeDtypeStruct((np_,), jnp.float32),
        ],
        mesh=mesh,
        scratch_types=[
            pltpu.VMEM((c * kw + _L,), jnp.int32),
            pltpu.VMEM((np_,), jnp.float32),
            pltpu.VMEM((glen, d), jnp.float32),
            pltpu.VMEM((glen, d), jnp.float32),
            pltpu.VMEM((glen, d), jnp.float32),
            pltpu.VMEM((c, d), jnp.float32),
            pltpu.VMEM((c,), jnp.float32),
            pltpu.SemaphoreType.DMA,
            pltpu.SemaphoreType.DMA,
            pltpu.SemaphoreType.DMA,
        ],
        compiler_params=pltpu.CompilerParams(needs_layout_passes=False),
    )


def kernel(feat_prop, neigh_idx, W, b):
    n, d = feat_prop.shape
    k = neigh_idx.shape[1]
    out_f = W.shape[0]
    c = (n + _NW - 1) // _NW
    c = (c + 7) // 8 * 8
    np_ = c * _NW

    kw = (k + 1 + 7) // 8 * 8
    xp = jnp.pad(feat_prop, ((0, np_ - n), (0, 0)))
    idxp = jnp.pad(neigh_idx.astype(jnp.int32), ((0, np_ - n), (0, 0)))
    idxe = jnp.concatenate(
        [idxp, jnp.arange(np_, dtype=jnp.int32)[:, None],
         jnp.zeros((np_, kw - k - 1), jnp.int32)], axis=1).reshape(np_ * kw)

    xhat, nrm = pl.pallas_call(
        _prep_body,
        out_shape=[
            jax.ShapeDtypeStruct((np_, d), jnp.float32),
            jax.ShapeDtypeStruct((np_,), jnp.float32),
        ],
    )(xp)

    s, wsum = _make_sc_gather(np_, c, k, d)(xhat, idxe, nrm)
    wsum = wsum.reshape(np_, 1)

    blk = 1024
    grid = np_ // blk
    out = pl.pallas_call(
        functools.partial(_final_body, 1.0 / k, d),
        grid=(grid,),
        in_specs=[
            pl.BlockSpec((blk, d), lambda i: (i, 0)),
            pl.BlockSpec((blk, 1), lambda i: (i, 0)),
            pl.BlockSpec((blk, d), lambda i: (i, 0)),
            pl.BlockSpec((out_f, 2 * d), lambda i: (0, 0)),
            pl.BlockSpec((out_f,), lambda i: (0,)),
        ],
        out_specs=pl.BlockSpec((blk, out_f), lambda i: (i, 0)),
        out_shape=jax.ShapeDtypeStruct((np_, out_f), jnp.float32),
    )(s, wsum, xp, W, b)

    return out[:n]


# speedup vs baseline: 2.9247x; 2.9247x over previous
import functools

import jax
import jax.numpy as jnp
from jax import lax
from jax.experimental import pallas as pl
from jax.experimental.pallas import tpu as pltpu
from jax.experimental.pallas import tpu_sc as plsc

_NC = 2
_NS = 16
_NW = _NC * _NS
_L = 16


def _prep_body(x_ref, xh_ref, nrm_ref):
    x = x_ref[...]
    ss = jnp.sum(x * x, axis=1, keepdims=True)
    r = lax.rsqrt(ss)
    xh_ref[...] = x * r
    nrm_ref[...] = (ss * r)[:, 0]


def _final_body(k_inv, d, s_ref, ws_ref, x_ref, w_ref, b_ref, o_ref):
    wn = w_ref[:, :d]
    wc = w_ref[:, d:]
    dn = (((1,), (1,)), ((), ()))
    ctr = lax.dot_general(x_ref[...], wc, dn,
                          preferred_element_type=jnp.float32) + b_ref[...]
    sn = lax.dot_general(s_ref[...], wn, dn,
                         preferred_element_type=jnp.float32)
    o_ref[...] = (sn + ws_ref[...] * ctr) * k_inv


def _make_sc_gather(np_, c, k, d):
    mesh = plsc.VectorSubcoreMesh(core_axis_name="c", subcore_axis_name="s")
    nsub = d // _L
    kw = (k + 1 + 7) // 8 * 8
    ng = (k + 1 + _L - 1) // _L
    G = 1
    span = G * kw
    glen = (G - 1) * kw + k + 1
    NBUF = 3
    nblk = c // G

    def body(xh_hbm, idx_hbm, nrm_hbm, s_hbm, ws_hbm,
             idx_v, nrm_v, b0, b1, b2, s_v, ws_v, sem0, sem1, sem2):
        bufs = (b0, b1, b2)
        sems = (sem0, sem1, sem2)
        wid = lax.axis_index("s") * _NC + lax.axis_index("c")
        base = wid * c
        pltpu.sync_copy(idx_hbm.at[pl.ds(base * kw, c * kw)],
                        idx_v.at[pl.ds(0, c * kw)])
        idx_v[pl.ds(c * kw, _L)] = jnp.zeros((_L,), jnp.int32)
        pltpu.sync_copy(nrm_hbm, nrm_v)
        lane = lax.iota(jnp.int32, _L)

        def start(blk, buf, sem):
            pltpu.async_copy(
                xh_hbm.at[idx_v.at[pl.ds(blk * span, glen)]], buf, sem)

        def wait(buf, sem):
            pltpu.make_async_copy(
                xh_hbm.at[idx_v.at[pl.ds(0, glen)]], buf, sem).wait()

        def compute(blk, buf):
            ib = blk * span
            for g in range(G):
                r0 = g * kw
                xn = [buf[r0 + k, pl.ds(t * _L, _L)] for t in range(nsub)]
                acc = [jnp.zeros((_L,), jnp.float32) for _ in range(nsub)]
                ws = jnp.float32(0.0)
                for grp in range(k // _L):
                    rng = plsc.load_gather(
                        nrm_v, [idx_v[pl.ds(ib + g * kw + grp * _L, _L)]])
                    rb = r0 + grp * _L

                    def edge(e2, carry):
                        *acc2, ws2 = carry
                        row = [buf[rb + e2, pl.ds(t * _L, _L)]
                               for t in range(nsub)]
                        dv = row[0] * xn[0]
                        for t in range(1, nsub):
                            dv = dv + row[t] * xn[t]
                        w = jnp.sum(dv)
                        wn = w * rng.at[jnp.full((_L,), e2, jnp.int32)].get(
                            mode="promise_in_bounds")
                        return tuple(
                            [acc2[t] + wn * row[t] for t in range(nsub)]
                            + [ws2 + w])
                    *acc, ws = lax.fori_loop(0, _L, edge, (*acc, ws),
                                             unroll=2)
                i = blk * G + g
                for t in range(nsub):
                    s_v[i, pl.ds(t * _L, _L)] = acc[t]
                plsc.store_scatter(ws_v, [jnp.full((_L,), i, jnp.int32)],
                                   jnp.full((_L,), ws, jnp.float32),
                                   mask=lane == 0)

        def blk_body(blk, carry):
            pltpu.async_copy(
                xh_hbm.at[idx_v.at[pl.ds(blk * span, glen)]],
                bufs[0], sems[0]).wait()
            compute(blk, bufs[0])
            return carry

        lax.fori_loop(0, nblk, blk_body, 0, unroll=False)
        pltpu.sync_copy(s_v, s_hbm.at[pl.ds(base, c)])
        pltpu.sync_copy(ws_v, ws_hbm.at[pl.ds(base, c)])

    return pl.kernel(
        body,
        out_type=[
            jax.ShapeDtypeStruct((np_, d), jnp.float32),
            jax.ShapeDtypeStruct((np_,), jnp.float32),
        ],
        mesh=mesh,
        scratch_types=[
            pltpu.VMEM((c * kw + _L,), jnp.int32),
            pltpu.VMEM((np_,), jnp.float32),
            pltpu.VMEM((glen, d), jnp.float32),
            pltpu.VMEM((glen, d), jnp.float32),
            pltpu.VMEM((glen, d), jnp.float32),
            pltpu.VMEM((c, d), jnp.float32),
            pltpu.VMEM((c,), jnp.float32),
            pltpu.SemaphoreType.DMA,
            pltpu.SemaphoreType.DMA,
            pltpu.SemaphoreType.DMA,
        ],
        compiler_params=pltpu.CompilerParams(needs_layout_passes=False),
    )


def kernel(feat_prop, neigh_idx, W, b):
    n, d = feat_prop.shape
    k = neigh_idx.shape[1]
    out_f = W.shape[0]
    c = (n + _NW - 1) // _NW
    c = (c + 7) // 8 * 8
    np_ = c * _NW

    kw = (k + 1 + 7) // 8 * 8
    xp = jnp.pad(feat_prop, ((0, np_ - n), (0, 0)))
    idxp = jnp.pad(neigh_idx.astype(jnp.int32), ((0, np_ - n), (0, 0)))
    idxe = jnp.concatenate(
        [idxp, jnp.arange(np_, dtype=jnp.int32)[:, None],
         jnp.zeros((np_, kw - k - 1), jnp.int32)], axis=1).reshape(np_ * kw)

    xhat, nrm = pl.pallas_call(
        _prep_body,
        out_shape=[
            jax.ShapeDtypeStruct((np_, d), jnp.float32),
            jax.ShapeDtypeStruct((np_,), jnp.float32),
        ],
    )(xp)

    s, wsum = _make_sc_gather(np_, c, k, d)(xhat, idxe, nrm)
    wsum = wsum.reshape(np_, 1)

    blk = 1024
    grid = np_ // blk
    out = pl.pallas_call(
        functools.partial(_final_body, 1.0 / k, d),
        grid=(grid,),
        in_specs=[
            pl.BlockSpec((blk, d), lambda i: (i, 0)),
            pl.BlockSpec((blk, 1), lambda i: (i, 0)),
            pl.BlockSpec((blk, d), lambda i: (i, 0)),
            pl.BlockSpec((out_f, 2 * d), lambda i: (0, 0)),
            pl.BlockSpec((out_f,), lambda i: (0,)),
        ],
        out_specs=pl.BlockSpec((blk, out_f), lambda i: (i, 0)),
        out_shape=jax.ShapeDtypeStruct((np_, out_f), jnp.float32),
    )(s, wsum, xp, W, b)

    return out[:n]


# speedup vs baseline: 4.0810x; 1.3954x over previous
import functools

import jax
import jax.numpy as jnp
from jax import lax
from jax.experimental import pallas as pl
from jax.experimental.pallas import tpu as pltpu
from jax.experimental.pallas import tpu_sc as plsc

_NC = 2
_NS = 16
_NW = _NC * _NS
_L = 16


def _prep_body(x_ref, xh_ref, nrm_ref):
    x = x_ref[...]
    ss = jnp.sum(x * x, axis=1, keepdims=True)
    r = lax.rsqrt(ss)
    xh_ref[...] = x * r
    nrm_ref[...] = (ss * r)[:, 0]


def _final_body(k_inv, d, s_ref, ws_ref, x_ref, w_ref, b_ref, o_ref):
    wn = w_ref[:, :d]
    wc = w_ref[:, d:]
    dn = (((1,), (1,)), ((), ()))
    ctr = lax.dot_general(x_ref[...], wc, dn,
                          preferred_element_type=jnp.float32) + b_ref[...]
    sn = lax.dot_general(s_ref[...], wn, dn,
                         preferred_element_type=jnp.float32)
    o_ref[...] = (sn + ws_ref[...] * ctr) * k_inv


def _make_sc_gather(np_, c, k, d):
    mesh = plsc.VectorSubcoreMesh(core_axis_name="c", subcore_axis_name="s")
    nsub = d // _L
    kw = (k + 1 + 7) // 8 * 8
    ng = (k + 1 + _L - 1) // _L
    G = 1
    span = G * kw
    glen = (G - 1) * kw + k + 1
    NBUF = 4
    nblk = c // G

    def body(xh_hbm, idx_hbm, nrm_hbm, s_hbm, ws_hbm,
             idx_v, nrm_v, b0, b1, b2, b3, s_v, ws_v,
             sem0, sem1, sem2, sem3):
        bufs = (b0, b1, b2, b3)
        sems = (sem0, sem1, sem2, sem3)
        wid = lax.axis_index("s") * _NC + lax.axis_index("c")
        base = wid * c
        pltpu.sync_copy(idx_hbm.at[pl.ds(base * kw, c * kw)],
                        idx_v.at[pl.ds(0, c * kw)])
        idx_v[pl.ds(c * kw, _L)] = jnp.zeros((_L,), jnp.int32)
        pltpu.sync_copy(nrm_hbm, nrm_v)
        lane = lax.iota(jnp.int32, _L)

        def start(blk, buf, sem):
            pltpu.async_copy(
                xh_hbm.at[idx_v.at[pl.ds(blk * span, glen)]], buf, sem)

        def wait(buf, sem):
            pltpu.make_async_copy(
                xh_hbm.at[idx_v.at[pl.ds(0, glen)]], buf, sem).wait()

        def compute(blk, buf):
            ib = blk * span
            for g in range(G):
                r0 = g * kw
                xn = [buf[r0 + k, pl.ds(t * _L, _L)] for t in range(nsub)]
                acc = [jnp.zeros((_L,), jnp.float32) for _ in range(nsub)]
                ws = jnp.float32(0.0)
                for grp in range(k // _L):
                    rng = plsc.load_gather(
                        nrm_v, [idx_v[pl.ds(ib + g * kw + grp * _L, _L)]])
                    rb = r0 + grp * _L

                    def edge(e2, carry):
                        *acc2, ws2 = carry
                        row = [buf[rb + e2, pl.ds(t * _L, _L)]
                               for t in range(nsub)]
                        dv = row[0] * xn[0]
                        for t in range(1, nsub):
                            dv = dv + row[t] * xn[t]
                        w = jnp.sum(dv)
                        wn = w * rng.at[jnp.full((_L,), e2, jnp.int32)].get(
                            mode="promise_in_bounds")
                        return tuple(
                            [acc2[t] + wn * row[t] for t in range(nsub)]
                            + [ws2 + w])
                    *acc, ws = lax.fori_loop(0, _L, edge, (*acc, ws),
                                             unroll=2)
                i = blk * G + g
                for t in range(nsub):
                    s_v[i, pl.ds(t * _L, _L)] = acc[t]
                plsc.store_scatter(ws_v, [jnp.full((_L,), i, jnp.int32)],
                                   jnp.full((_L,), ws, jnp.float32),
                                   mask=lane == 0)

        for j in range(NBUF - 1):
            start(jnp.int32(j), bufs[j], sems[j])

        def blk_grp(ii, carry):
            for j in range(NBUF):
                blk = ii * NBUF + j

                @pl.when(blk < nblk)
                def _():
                    wait(bufs[j], sems[j])

                    @pl.when(blk + NBUF - 1 < nblk)
                    def _():
                        start(blk + NBUF - 1, bufs[(j + NBUF - 1) % NBUF],
                              sems[(j + NBUF - 1) % NBUF])

                    compute(blk, bufs[j])
            return carry

        ngrp = (nblk + NBUF - 1) // NBUF
        lax.fori_loop(0, ngrp, blk_grp, 0, unroll=False)
        pltpu.sync_copy(s_v, s_hbm.at[pl.ds(base, c)])
        pltpu.sync_copy(ws_v, ws_hbm.at[pl.ds(base, c)])

    return pl.kernel(
        body,
        out_type=[
            jax.ShapeDtypeStruct((np_, d), jnp.float32),
            jax.ShapeDtypeStruct((np_,), jnp.float32),
        ],
        mesh=mesh,
        scratch_types=[
            pltpu.VMEM((c * kw + _L,), jnp.int32),
            pltpu.VMEM((np_,), jnp.float32),
            pltpu.VMEM((glen, d), jnp.float32),
            pltpu.VMEM((glen, d), jnp.float32),
            pltpu.VMEM((glen, d), jnp.float32),
            pltpu.VMEM((glen, d), jnp.float32),
            pltpu.VMEM((c, d), jnp.float32),
            pltpu.VMEM((c,), jnp.float32),
            pltpu.SemaphoreType.DMA,
            pltpu.SemaphoreType.DMA,
            pltpu.SemaphoreType.DMA,
            pltpu.SemaphoreType.DMA,
        ],
        compiler_params=pltpu.CompilerParams(needs_layout_passes=False),
    )


def kernel(feat_prop, neigh_idx, W, b):
    n, d = feat_prop.shape
    k = neigh_idx.shape[1]
    out_f = W.shape[0]
    c = (n + _NW - 1) // _NW
    c = (c + 7) // 8 * 8
    np_ = c * _NW

    kw = (k + 1 + 7) // 8 * 8
    xp = jnp.pad(feat_prop, ((0, np_ - n), (0, 0)))
    idxp = jnp.pad(neigh_idx.astype(jnp.int32), ((0, np_ - n), (0, 0)))
    idxe = jnp.concatenate(
        [idxp, jnp.arange(np_, dtype=jnp.int32)[:, None],
         jnp.zeros((np_, kw - k - 1), jnp.int32)], axis=1).reshape(np_ * kw)

    xhat, nrm = pl.pallas_call(
        _prep_body,
        out_shape=[
            jax.ShapeDtypeStruct((np_, d), jnp.float32),
            jax.ShapeDtypeStruct((np_,), jnp.float32),
        ],
    )(xp)

    s, wsum = _make_sc_gather(np_, c, k, d)(xhat, idxe, nrm)
    wsum = wsum.reshape(np_, 1)

    blk = 1024
    grid = np_ // blk
    out = pl.pallas_call(
        functools.partial(_final_body, 1.0 / k, d),
        grid=(grid,),
        in_specs=[
            pl.BlockSpec((blk, d), lambda i: (i, 0)),
            pl.BlockSpec((blk, 1), lambda i: (i, 0)),
            pl.BlockSpec((blk, d), lambda i: (i, 0)),
            pl.BlockSpec((out_f, 2 * d), lambda i: (0, 0)),
            pl.BlockSpec((out_f,), lambda i: (0,)),
        ],
        out_specs=pl.BlockSpec((blk, out_f), lambda i: (i, 0)),
        out_shape=jax.ShapeDtypeStruct((np_, out_f), jnp.float32),
    )(s, wsum, xp, W, b)

    return out[:n]
